# baseline (speedup 1.0000x reference)
import functools

import jax
import jax.numpy as jnp
from jax import lax
from jax.experimental import pallas as pl
from jax.experimental.pallas import tpu as pltpu
from jax.experimental.pallas import tpu_sc as plsc

_T, _B, _C = 512, 64, 1024
_TBLK = 16
_BLANK = 0

_NC, _NS, _L = 2, 16, 16
_NW = _NC * _NS
_BPW = _B // _NW
_PAD = _L


def _greedy_body(dlen_ref, x_ref, lp_ref, len_ref, pos_ref, seq_ref,
                 s_ref, eacc_ref):
    i = pl.program_id(0)

    @pl.when(i == 0)
    def _init():
        s_ref[...] = jnp.zeros((1, _B), jnp.float32)

    x = x_ref[...]
    m = jnp.max(x, axis=-1)
    dl = dlen_ref[...]
    t0 = i * _TBLK
    s = s_ref[...]
    rows = []
    for j in range(_TBLK):
        rows.append(s)
        active = (t0 + j) < dl
        s = jnp.where(active, s + m[j:j + 1, :], s)
    s_ref[...] = s
    sexc = jnp.concatenate(rows, axis=0)
    v = sexc[..., None] + x
    vm = sexc + m
    c_iota = lax.broadcasted_iota(jnp.int32, v.shape, 2)
    eacc_ref[pl.ds(t0, _TBLK), :] = jnp.min(
        jnp.where(v == vm[..., None], c_iota, _C), axis=-1)

    @pl.when(i == (_T // _TBLK) - 1)
    def _collapse():
        lp_ref[...] = -s_ref[...]
        e = eacc_ref[...].T
        dlc = dl.reshape(_B, 1)
        t_i = lax.broadcasted_iota(jnp.int32, (_B, _T), 1)
        active = t_i < dlc
        seq = jnp.where(active, e, _BLANK)
        prev = jnp.concatenate(
            [jnp.full((_B, 1), -1, jnp.int32), seq[:, :-1]], axis=1)
        keep = active & (seq != _BLANK) & (seq != prev)
        keepf = keep.astype(jnp.float32)
        r = lax.broadcasted_iota(jnp.int32, (_T, _T), 0)
        c = lax.broadcasted_iota(jnp.int32, (_T, _T), 1)
        tri = (r <= c).astype(jnp.float32)
        cs = jax.lax.dot(keepf, tri, preferred_element_type=jnp.float32)
        cs_i = cs.astype(jnp.int32)
        len_ref[...] = cs_i[:, -1:]
        sentinel = _T + (t_i & (_L - 1))
        pos_ref[...] = jnp.where(keep, cs_i - 1, sentinel)
        seq_ref[...] = seq


def _scatter_body(pos_hbm, seq_hbm, dec_hbm, pos_v, seq_v, row_v):
    wid = lax.axis_index("s") * _NC + lax.axis_index("c")
    neg1 = jnp.full((_L,), -1, jnp.int32)
    for k in range(_BPW):
        b = wid * _BPW + k
        pltpu.sync_copy(pos_hbm.at[pl.ds(b * _T, _T)], pos_v)
        pltpu.sync_copy(seq_hbm.at[pl.ds(b * _T, _T)], seq_v)
        for j in range((_T + _PAD) // _L):
            row_v[pl.ds(j * _L, _L)] = neg1
        for j in range(_T // _L):
            p = pos_v[pl.ds(j * _L, _L)]
            s = seq_v[pl.ds(j * _L, _L)]
            plsc.store_scatter(row_v, [p], s, mask=p < _T)
        pltpu.sync_copy(row_v.at[pl.ds(0, _T)], dec_hbm.at[pl.ds(b * _T, _T)])


@functools.cache
def _scatter_call():
    return pl.kernel(
        _scatter_body,
        out_type=jax.ShapeDtypeStruct((_B * _T,), jnp.int32),
        mesh=plsc.VectorSubcoreMesh(core_axis_name="c", subcore_axis_name="s",
                                    num_cores=_NC, num_subcores=_NS),
        compiler_params=pltpu.CompilerParams(needs_layout_passes=False),
        scratch_types=[
            pltpu.VMEM((_T,), jnp.int32),
            pltpu.VMEM((_T,), jnp.int32),
            pltpu.VMEM((_T + _PAD,), jnp.int32),
        ],
    )


def _greedy_call(data, dlen_row, interpret=False):
    T, B, C = data.shape
    return pl.pallas_call(
        _greedy_body,
        grid=(T // _TBLK,),
        in_specs=[
            pl.BlockSpec((1, B), lambda i: (0, 0)),
            pl.BlockSpec((_TBLK, B, C), lambda i: (i, 0, 0)),
        ],
        out_specs=[
            pl.BlockSpec((1, B), lambda i: (0, 0)),
            pl.BlockSpec((B, 1), lambda i: (0, 0)),
            pl.BlockSpec((B, T), lambda i: (0, 0)),
            pl.BlockSpec((B, T), lambda i: (0, 0)),
        ],
        out_shape=[
            jax.ShapeDtypeStruct((1, B), jnp.float32),
            jax.ShapeDtypeStruct((B, 1), jnp.int32),
            jax.ShapeDtypeStruct((B, T), jnp.int32),
            jax.ShapeDtypeStruct((B, T), jnp.int32),
        ],
        scratch_shapes=[
            pltpu.VMEM((1, B), jnp.float32),
            pltpu.VMEM((T, B), jnp.int32),
        ],
        interpret=interpret,
    )(dlen_row, data)


def kernel(data, data_length):
    T, B, C = data.shape
    dlen = data_length.astype(jnp.int32)
    lp_row, lens, pos, seq = _greedy_call(data, dlen[None, :])
    dec = _scatter_call()(pos.reshape(-1), seq.reshape(-1))
    decoded = dec.reshape(B, T)[:, None, :]
    return lp_row.reshape(B, 1), lens, decoded


# speedup vs baseline: 1.0744x; 1.0744x over previous
import functools

import jax
import jax.numpy as jnp
from jax import lax
from jax.experimental import pallas as pl
from jax.experimental.pallas import tpu as pltpu
from jax.experimental.pallas import tpu_sc as plsc

_T, _B, _C = 512, 64, 1024
_TBLK = 32
_BLANK = 0

_NC, _NS, _L = 2, 16, 16
_NW = _NC * _NS
_BPW = _B // _NW
_PAD = _L


def _greedy_body(dlen_ref, x_ref, lp_ref, len_ref, pos_ref, seq_ref,
                 s_ref, eacc_ref):
    i = pl.program_id(0)

    @pl.when(i == 0)
    def _init():
        s_ref[...] = jnp.zeros((1, _B), jnp.float32)

    x = x_ref[...]
    m = jnp.max(x, axis=-1)
    dl = dlen_ref[...]
    t0 = i * _TBLK
    s = s_ref[...]
    rows = []
    for j in range(_TBLK):
        rows.append(s)
        active = (t0 + j) < dl
        s = jnp.where(active, s + m[j:j + 1, :], s)
    s_ref[...] = s
    sexc = jnp.concatenate(rows, axis=0)
    v = sexc[..., None] + x
    vm = sexc + m
    c_iota = lax.broadcasted_iota(jnp.int32, v.shape, 2)
    eacc_ref[pl.ds(t0, _TBLK), :] = jnp.min(
        jnp.where(v == vm[..., None], c_iota, _C), axis=-1)

    @pl.when(i == (_T // _TBLK) - 1)
    def _collapse():
        lp_ref[...] = -s_ref[...]
        e = eacc_ref[...].T
        dlc = dl.reshape(_B, 1)
        t_i = lax.broadcasted_iota(jnp.int32, (_B, _T), 1)
        active = t_i < dlc
        seq = jnp.where(active, e, _BLANK)
        prev = jnp.concatenate(
            [jnp.full((_B, 1), -1, jnp.int32), seq[:, :-1]], axis=1)
        keep = active & (seq != _BLANK) & (seq != prev)
        keepf = keep.astype(jnp.float32)
        r = lax.broadcasted_iota(jnp.int32, (_T, _T), 0)
        c = lax.broadcasted_iota(jnp.int32, (_T, _T), 1)
        tri = (r <= c).astype(jnp.float32)
        cs = jax.lax.dot(keepf, tri, preferred_element_type=jnp.float32)
        cs_i = cs.astype(jnp.int32)
        len_ref[...] = cs_i[:, -1:]
        sentinel = _T + (t_i & (_L - 1))
        pos_ref[...] = jnp.where(keep, cs_i - 1, sentinel)
        seq_ref[...] = seq


def _scatter_body(pos_hbm, seq_hbm, dec_hbm, pos_v, seq_v, row_v):
    wid = lax.axis_index("s") * _NC + lax.axis_index("c")
    neg1 = jnp.full((_L,), -1, jnp.int32)
    for k in range(_BPW):
        b = wid * _BPW + k
        pltpu.sync_copy(pos_hbm.at[pl.ds(b * _T, _T)], pos_v)
        pltpu.sync_copy(seq_hbm.at[pl.ds(b * _T, _T)], seq_v)
        for j in range((_T + _PAD) // _L):
            row_v[pl.ds(j * _L, _L)] = neg1
        for j in range(_T // _L):
            p = pos_v[pl.ds(j * _L, _L)]
            s = seq_v[pl.ds(j * _L, _L)]
            plsc.store_scatter(row_v, [p], s, mask=p < _T)
        pltpu.sync_copy(row_v.at[pl.ds(0, _T)], dec_hbm.at[pl.ds(b * _T, _T)])


@functools.cache
def _scatter_call():
    return pl.kernel(
        _scatter_body,
        out_type=jax.ShapeDtypeStruct((_B * _T,), jnp.int32),
        mesh=plsc.VectorSubcoreMesh(core_axis_name="c", subcore_axis_name="s",
                                    num_cores=_NC, num_subcores=_NS),
        compiler_params=pltpu.CompilerParams(needs_layout_passes=False),
        scratch_types=[
            pltpu.VMEM((_T,), jnp.int32),
            pltpu.VMEM((_T,), jnp.int32),
            pltpu.VMEM((_T + _PAD,), jnp.int32),
        ],
    )


def _greedy_call(data, dlen_row, interpret=False):
    T, B, C = data.shape
    return pl.pallas_call(
        _greedy_body,
        grid=(T // _TBLK,),
        in_specs=[
            pl.BlockSpec((1, B), lambda i: (0, 0)),
            pl.BlockSpec((_TBLK, B, C), lambda i: (i, 0, 0)),
        ],
        out_specs=[
            pl.BlockSpec((1, B), lambda i: (0, 0)),
            pl.BlockSpec((B, 1), lambda i: (0, 0)),
            pl.BlockSpec((B, T), lambda i: (0, 0)),
            pl.BlockSpec((B, T), lambda i: (0, 0)),
        ],
        out_shape=[
            jax.ShapeDtypeStruct((1, B), jnp.float32),
            jax.ShapeDtypeStruct((B, 1), jnp.int32),
            jax.ShapeDtypeStruct((B, T), jnp.int32),
            jax.ShapeDtypeStruct((B, T), jnp.int32),
        ],
        scratch_shapes=[
            pltpu.VMEM((1, B), jnp.float32),
            pltpu.VMEM((T, B), jnp.int32),
        ],
        interpret=interpret,
    )(dlen_row, data)


def kernel(data, data_length):
    T, B, C = data.shape
    dlen = data_length.astype(jnp.int32)
    lp_row, lens, pos, seq = _greedy_call(data, dlen[None, :])
    dec = _scatter_call()(pos.reshape(-1), seq.reshape(-1))
    decoded = dec.reshape(B, T)[:, None, :]
    return lp_row.reshape(B, 1), lens, decoded
